# Initial kernel scaffold; baseline (speedup 1.0000x reference)
import jax
import jax.numpy as jnp
from jax.experimental import pallas as pl


def kernel(x, mesh_x, g2m_efeat, mesh_efeat, m2g_efeat, g2m_src, g2m_dst, mesh_src, mesh_dst, m2g_src, m2g_dst, params):
    raise NotImplementedError("write your pallas kernel here")


# speedup vs baseline: 1.6587x; 1.6587x over previous
import functools

import jax
import jax.numpy as jnp
from jax import lax
from jax.experimental import pallas as pl
from jax.experimental.pallas import tpu as pltpu
from jax.experimental.pallas import tpu_sc as plsc

N_GRID = 64800
N_MESH = 10242
M_PAD = 10368
W, H = 180, 360
HID = 128
F32 = jnp.float32

NC, NS = 2, 16
NW = NC * NS


def _ln(h, g, b):
    mu = jnp.mean(h, axis=-1, keepdims=True)
    var = jnp.mean((h - mu) ** 2, axis=-1, keepdims=True)
    return (h - mu) * lax.rsqrt(var + 1e-5) * g + b


def _full(shape):
    return pl.BlockSpec(shape, lambda i: (0,) * len(shape))


def _rows(bsize, ncol):
    return pl.BlockSpec((bsize, ncol), lambda i: (i, 0))


def _vec(v):
    return v.reshape(1, -1)


def _embed(x, p, bsize):
    n, din = x.shape
    dp = -(-din // 8) * 8
    if dp != din:
        x = jnp.pad(x, ((0, 0), (0, dp - din)))
        w1 = jnp.pad(p["w1"], ((0, dp - din), (0, 0)))
    else:
        w1 = p["w1"]

    def kern(x_ref, w1_ref, b1_ref, w2_ref, b2_ref, g_ref, bb_ref, o_ref):
        h = jax.nn.silu(
            jnp.dot(x_ref[...], w1_ref[...], preferred_element_type=F32)
            + b1_ref[...])
        h = jnp.dot(h, w2_ref[...], preferred_element_type=F32) + b2_ref[...]
        o_ref[...] = _ln(h, g_ref[...], bb_ref[...])

    return pl.pallas_call(
        kern,
        grid=(n // bsize,),
        in_specs=[_rows(bsize, dp), _full((dp, HID)), _full((1, HID)),
                  _full((HID, HID)), _full((1, HID)), _full((1, HID)),
                  _full((1, HID))],
        out_specs=_rows(bsize, HID),
        out_shape=jax.ShapeDtypeStruct((n, HID), F32),
    )(x, w1, _vec(p["b1"]), p["w2"], _vec(p["b2"]), _vec(p["ln_g"]),
      _vec(p["ln_b"]))


def _res_mlp(x, p, bsize):
    n = x.shape[0]

    def kern(x_ref, w1_ref, b1_ref, w2_ref, b2_ref, g_ref, bb_ref, o_ref):
        xv = x_ref[...]
        h = jax.nn.silu(
            jnp.dot(xv, w1_ref[...], preferred_element_type=F32) + b1_ref[...])
        h = jnp.dot(h, w2_ref[...], preferred_element_type=F32) + b2_ref[...]
        o_ref[...] = _ln(h, g_ref[...], bb_ref[...]) + xv

    return pl.pallas_call(
        kern,
        grid=(n // bsize,),
        in_specs=[_rows(bsize, HID), _full((HID, HID)), _full((1, HID)),
                  _full((HID, HID)), _full((1, HID)), _full((1, HID)),
                  _full((1, HID))],
        out_specs=_rows(bsize, HID),
        out_shape=jax.ShapeDtypeStruct((n, HID), F32),
    )(x, p["w1"], _vec(p["b1"]), p["w2"], _vec(p["b2"]), _vec(p["ln_g"]),
      _vec(p["ln_b"]))


def _proj(x, w, bsize):
    n = x.shape[0]

    def kern(x_ref, w_ref, o_ref):
        o_ref[...] = jnp.dot(x_ref[...], w_ref[...], preferred_element_type=F32)

    return pl.pallas_call(
        kern,
        grid=(n // bsize,),
        in_specs=[_rows(bsize, HID), _full((HID, HID))],
        out_specs=_rows(bsize, HID),
        out_shape=jax.ShapeDtypeStruct((n, HID), F32),
    )(x, w)


def _proj2(x, wa, wb, bsize):
    n = x.shape[0]

    def kern(x_ref, wa_ref, wb_ref, oa_ref, ob_ref):
        xv = x_ref[...]
        oa_ref[...] = jnp.dot(xv, wa_ref[...], preferred_element_type=F32)
        ob_ref[...] = jnp.dot(xv, wb_ref[...], preferred_element_type=F32)

    return pl.pallas_call(
        kern,
        grid=(n // bsize,),
        in_specs=[_rows(bsize, HID), _full((HID, HID)), _full((HID, HID))],
        out_specs=[_rows(bsize, HID), _rows(bsize, HID)],
        out_shape=[jax.ShapeDtypeStruct((n, HID), F32),
                   jax.ShapeDtypeStruct((n, HID), F32)],
    )(x, wa, wb)


def _edge_stage(e, gsum, p, bsize):
    n = e.shape[0]
    w1a = p["w1"][:HID]

    def kern(e_ref, gs_ref, w1_ref, b1_ref, w2_ref, b2_ref, g_ref, bb_ref,
             o_ref):
        ev = e_ref[...]
        pre = (jnp.dot(ev, w1_ref[...], preferred_element_type=F32)
               + gs_ref[...] + b1_ref[...])
        h = (jnp.dot(jax.nn.silu(pre), w2_ref[...], preferred_element_type=F32)
             + b2_ref[...])
        o_ref[...] = _ln(h, g_ref[...], bb_ref[...]) + ev

    return pl.pallas_call(
        kern,
        grid=(n // bsize,),
        in_specs=[_rows(bsize, HID), _rows(bsize, HID), _full((HID, HID)),
                  _full((1, HID)), _full((HID, HID)), _full((1, HID)),
                  _full((1, HID)), _full((1, HID))],
        out_specs=_rows(bsize, HID),
        out_shape=jax.ShapeDtypeStruct((n, HID), F32),
    )(e, gsum, w1a, _vec(p["b1"]), p["w2"], _vec(p["b2"]), _vec(p["ln_g"]),
      _vec(p["ln_b"]))


def _node_stage(aggs, node, p, bsize):
    n = node.shape[0]
    nagg = len(aggs)
    w1a = p["w1"][:HID]
    w1b = p["w1"][HID:]

    def kern(*refs):
        agg_refs = refs[:nagg]
        (node_ref, w1a_ref, w1b_ref, b1_ref, w2_ref, b2_ref, g_ref, bb_ref,
         o_ref) = refs[nagg:]
        a = agg_refs[0][...]
        for r in agg_refs[1:]:
            a = a + r[...]
        nv = node_ref[...]
        pre = (jnp.dot(a, w1a_ref[...], preferred_element_type=F32)
               + jnp.dot(nv, w1b_ref[...], preferred_element_type=F32)
               + b1_ref[...])
        h = (jnp.dot(jax.nn.silu(pre), w2_ref[...], preferred_element_type=F32)
             + b2_ref[...])
        o_ref[...] = _ln(h, g_ref[...], bb_ref[...]) + nv

    return pl.pallas_call(
        kern,
        grid=(n // bsize,),
        in_specs=[_rows(bsize, HID)] * nagg
        + [_rows(bsize, HID), _full((HID, HID)), _full((HID, HID)),
           _full((1, HID)), _full((HID, HID)), _full((1, HID)),
           _full((1, HID)), _full((1, HID))],
        out_specs=_rows(bsize, HID),
        out_shape=jax.ShapeDtypeStruct((n, HID), F32),
    )(*aggs, node, w1a, w1b, _vec(p["b1"]), p["w2"], _vec(p["b2"]),
      _vec(p["ln_g"]), _vec(p["ln_b"]))


def _decoder(x, p, bsize):
    n = x.shape[0]
    w2 = jnp.pad(p["w2"], ((0, 0), (0, 7)))
    b2 = jnp.pad(p["b2"], ((0, 7),))

    def kern(x_ref, w1_ref, b1_ref, w2_ref, b2_ref, o_ref):
        h = jax.nn.silu(
            jnp.dot(x_ref[...], w1_ref[...], preferred_element_type=F32)
            + b1_ref[...])
        o_ref[...] = (jnp.dot(h, w2_ref[...], preferred_element_type=F32)
                      + b2_ref[...])

    return pl.pallas_call(
        kern,
        grid=(n // bsize,),
        in_specs=[_rows(bsize, HID), _full((HID, HID)), _full((1, HID)),
                  _full((HID, 8)), _full((1, 8))],
        out_specs=_rows(bsize, 8),
        out_shape=jax.ShapeDtypeStruct((n, 8), F32),
    )(x, p["w1"], _vec(p["b1"]), w2, _vec(b2))


def _sc_gather_sum(tab_a, tab_b, idx_a, idx_b, chunk):
    ne = idx_a.shape[0]
    nchunks = ne // chunk
    iters = -(-nchunks // NW)
    mesh = plsc.VectorSubcoreMesh(core_axis_name="c", subcore_axis_name="s")

    @functools.partial(
        pl.kernel,
        out_type=jax.ShapeDtypeStruct((ne, HID), F32),
        mesh=mesh,
        scratch_types=[
            pltpu.VMEM((chunk,), jnp.int32),
            pltpu.VMEM((chunk,), jnp.int32),
            pltpu.VMEM((chunk, HID), F32),
            pltpu.VMEM((chunk, HID), F32),
            pltpu.SemaphoreType.DMA,
            pltpu.SemaphoreType.DMA,
        ],
    )
    def k(ta, tb, ia, ib, out, ia_v, ib_v, ba_v, bb_v, sa, sb):
        wid = lax.axis_index("s") * NC + lax.axis_index("c")

        def step(it, carry):
            cid = wid + it * NW

            @pl.when(cid < nchunks)
            def _():
                base = cid * chunk
                pltpu.sync_copy(ia.at[pl.ds(base, chunk)], ia_v)
                pltpu.sync_copy(ib.at[pl.ds(base, chunk)], ib_v)
                ca = pltpu.async_copy(ta.at[ia_v], ba_v, sa)
                cb = pltpu.async_copy(tb.at[ib_v], bb_v, sb)
                ca.wait()
                cb.wait()

                def addrow(i, c2):
                    for j in range(HID // 16):
                        sl = pl.ds(j * 16, 16)
                        ba_v[i, sl] = ba_v[i, sl] + bb_v[i, sl]
                    return c2

                lax.fori_loop(0, chunk, addrow, 0)
                pltpu.sync_copy(ba_v, out.at[pl.ds(base, chunk)])

            return carry

        lax.fori_loop(0, iters, step, 0)

    return k(tab_a, tab_b, idx_a, idx_b)


def _sc_segsum_mesh(vals, dst, chunk):
    ne = vals.shape[0]
    nchunks = ne // chunk
    iters = -(-(-(-nchunks // 2)) // NS)
    nzc = M_PAD // 128
    ziters = -(-nzc // NS)
    mesh = plsc.VectorSubcoreMesh(core_axis_name="c", subcore_axis_name="s")

    @functools.partial(
        pl.kernel,
        out_type=jax.ShapeDtypeStruct((2 * M_PAD, HID), F32),
        mesh=mesh,
        scratch_types=[
            pltpu.VMEM((chunk,), jnp.int32),
            pltpu.VMEM((chunk, HID), F32),
            pltpu.VMEM((128, HID), F32),
            pltpu.VMEM_SHARED((M_PAD, HID), F32),
        ],
    )
    def k(vals_h, dst_h, out_h, idx_v, buf_v, zb_v, acc_s):
        c = lax.axis_index("c")
        s = lax.axis_index("s")

        def zrow(i, c2):
            for j in range(HID // 16):
                zb_v[i, pl.ds(j * 16, 16)] = jnp.zeros((16,), F32)
            return c2

        lax.fori_loop(0, 128, zrow, 0)

        def zstep(it, c2):
            zc = s + it * NS

            @pl.when(zc < nzc)
            def _():
                pltpu.sync_copy(zb_v, acc_s.at[pl.ds(zc * 128, 128)])

            return c2

        lax.fori_loop(0, ziters, zstep, 0)
        plsc.subcore_barrier()

        def step(it, c2):
            cid = c + 2 * (s + it * NS)

            @pl.when(cid < nchunks)
            def _():
                base = cid * chunk
                pltpu.sync_copy(dst_h.at[pl.ds(base, chunk)], idx_v)
                pltpu.sync_copy(vals_h.at[pl.ds(base, chunk)], buf_v)
                pltpu.sync_copy(buf_v, acc_s.at[idx_v], add=True)

            return c2

        lax.fori_loop(0, iters, step, 0)
        plsc.subcore_barrier()

        def wstep(it, c2):
            zc = s + it * NS

            @pl.when(zc < nzc)
            def _():
                pltpu.sync_copy(acc_s.at[pl.ds(zc * 128, 128)], zb_v)
                pltpu.sync_copy(
                    zb_v, out_h.at[pl.ds(c * M_PAD + zc * 128, 128)])

            return c2

        lax.fori_loop(0, ziters, wstep, 0)

    return k(vals, dst)


_RANGE = 8128
_SP_ROWS = 8192
_DUMMY = 8128
_WB = 64
_NPASS = 4
_G_PAD = 2 * _NPASS * _RANGE


def _sc_segsum_grid(vals, dst, chunk):
    ne = vals.shape[0]
    nchunks = ne // chunk
    iters = -(-nchunks // NS)
    nzc = _SP_ROWS // 128
    ziters = -(-nzc // NS)
    nwb = _RANGE // _WB
    witers = -(-nwb // NS)
    mesh = plsc.VectorSubcoreMesh(core_axis_name="c", subcore_axis_name="s")

    @functools.partial(
        pl.kernel,
        out_type=jax.ShapeDtypeStruct((_G_PAD, HID), F32),
        mesh=mesh,
        scratch_types=[
            pltpu.VMEM((chunk,), jnp.int32),
            pltpu.VMEM((chunk,), jnp.int32),
            pltpu.VMEM((chunk, HID), F32),
            pltpu.VMEM((128, HID), F32),
            pltpu.VMEM((_WB, HID), F32),
            pltpu.VMEM_SHARED((_SP_ROWS, HID), F32),
        ],
    )
    def k(vals_h, dst_h, out_h, idx_v, rel_v, buf_v, zb_v, wb_v, acc_s):
        c = lax.axis_index("c")
        s = lax.axis_index("s")

        def zrow(i, c2):
            for j in range(HID // 16):
                zb_v[i, pl.ds(j * 16, 16)] = jnp.zeros((16,), F32)
            return c2

        lax.fori_loop(0, 128, zrow, 0)

        for p in range(_NPASS):
            rbase = (_NPASS * c + p) * _RANGE

            def zstep(it, c2):
                zc = s + it * NS

                @pl.when(zc < nzc)
                def _():
                    pltpu.sync_copy(zb_v, acc_s.at[pl.ds(zc * 128, 128)])

                return c2

            lax.fori_loop(0, ziters, zstep, 0)
            plsc.subcore_barrier()

            def step(it, c2):
                cid = s + it * NS

                @pl.when(cid < nchunks)
                def _():
                    base = cid * chunk
                    pltpu.sync_copy(dst_h.at[pl.ds(base, chunk)], idx_v)

                    def remap(i2, c3):
                        sl = pl.ds(i2 * 16, 16)
                        v = idx_v[sl] - rbase
                        ok = (v >= 0) & (v < _RANGE)
                        rel_v[sl] = jnp.where(ok, v, _DUMMY)
                        return c3

                    lax.fori_loop(0, chunk // 16, remap, 0)
                    pltpu.sync_copy(vals_h.at[pl.ds(base, chunk)], buf_v)
                    pltpu.sync_copy(buf_v, acc_s.at[rel_v], add=True)

                return c2

            lax.fori_loop(0, iters, step, 0)
            plsc.subcore_barrier()

            def wstep(it, c2):
                wc = s + it * NS

                @pl.when(wc < nwb)
                def _():
                    pltpu.sync_copy(acc_s.at[pl.ds(wc * _WB, _WB)], wb_v)
                    pltpu.sync_copy(
                        wb_v, out_h.at[pl.ds(rbase + wc * _WB, _WB)])

                return c2

            lax.fori_loop(0, witers, wstep, 0)
            plsc.subcore_barrier()

    return k(vals, dst)


def kernel(x, mesh_x, g2m_efeat, mesh_efeat, m2g_efeat,
           g2m_src, g2m_dst, mesh_src, mesh_dst, m2g_src, m2g_dst, params):
    p = params
    xf = x[:, 0, :, :].reshape(x.shape[0], -1).T

    grid = _embed(xf, p["grid_embed"], 480)
    mesh_xp = jnp.pad(mesh_x, ((0, M_PAD - N_MESH), (0, 0)))
    mesh_f = _embed(mesh_xp, p["mesh_embed"], 576)
    e_g2m = _embed(g2m_efeat, p["g2m_e_embed"], 640)
    e_mesh = _embed(mesh_efeat, p["mesh_e_embed"], 512)
    e_m2g = _embed(m2g_efeat, p["m2g_e_embed"], 480)

    ep = p["g2m_edge"]
    pg = _proj(grid, ep["w1"][HID:2 * HID], 480)
    pm = _proj(mesh_f, ep["w1"][2 * HID:], 576)
    gs = _sc_gather_sum(pg, pm, g2m_src, g2m_dst, 128)
    e_g2m = _edge_stage(e_g2m, gs, ep, 640)
    agg2 = _sc_segsum_mesh(e_g2m, g2m_dst, 128)
    mesh_f = _node_stage([agg2[:M_PAD], agg2[M_PAD:]], mesh_f,
                         p["g2m_node"], 576)
    grid = _res_mlp(grid, p["g2m_grid"], 480)

    for lp in p["proc"]:
        ew = lp["edge"]
        ps, pd = _proj2(mesh_f, ew["w1"][HID:2 * HID], ew["w1"][2 * HID:], 576)
        gs = _sc_gather_sum(ps, pd, mesh_src, mesh_dst, 128)
        e_mesh = _edge_stage(e_mesh, gs, ew, 512)
        agg2 = _sc_segsum_mesh(e_mesh, mesh_dst, 128)
        mesh_f = _node_stage([agg2[:M_PAD], agg2[M_PAD:]], mesh_f,
                             lp["node"], 576)

    ep = p["m2g_edge"]
    ps = _proj(mesh_f, ep["w1"][HID:2 * HID], 576)
    pd = _proj(grid, ep["w1"][2 * HID:], 480)
    gs = _sc_gather_sum(ps, pd, m2g_src, m2g_dst, 96)
    e = _edge_stage(e_m2g, gs, ep, 480)
    agg = _sc_segsum_grid(e, m2g_dst, 96)[:N_GRID]
    grid = _node_stage([agg], grid, p["m2g_node"], 480)

    out = _decoder(grid, p["dec"], 480)[:, :1]
    return out.reshape(W, H, 1).transpose(2, 0, 1)[None]
